# baseline (speedup 1.0000x reference)
import functools
import math

import numpy as np
import jax
import jax.numpy as jnp
from jax.experimental import pallas as pl
from jax.experimental.pallas import tpu as pltpu
from jax.experimental.pallas import tpu_sc as plsc

SINKHORN_ITERS = 8
EPS = 1e-6
N = 4096
E = 16
K = 256
NCOL = 64
LANES = 16
NCH = N // LANES
NBUCK = 2048
NBCH = NBUCK // LANES
CAP = 320
CAPB = CAP // LANES

_HALF_BITS = int(np.float32(math.log(0.5)).view(np.uint32))
_KEY_HALF_U = (~_HALF_BITS) & 0xFFFFFFFF
KEY_HALF = (_KEY_HALF_U ^ 0x80000000) - (1 << 32) if (_KEY_HALF_U ^ 0x80000000) >= (1 << 31) else (_KEY_HALF_U ^ 0x80000000)
KEY_PAD = -(2 ** 31)


def _lse_tokens(t):
    m = jnp.max(t, axis=1, keepdims=True)
    return m + jnp.log(jnp.sum(jnp.exp(t - m), axis=1, keepdims=True))


def _lse_experts(t):
    m = jnp.max(t, axis=0, keepdims=True)
    return m + jnp.log(jnp.sum(jnp.exp(t - m), axis=0, keepdims=True))


def _gates_kernel(x_ref, w_ref, tt_ref, acc_ref, *, nk):
    kk = pl.program_id(1)

    part = jax.lax.dot_general(
        x_ref[0], w_ref[0],
        dimension_numbers=(((1,), (0,)), ((), ())),
        preferred_element_type=jnp.float32,
    )

    @pl.when(kk == 0)
    def _init():
        acc_ref[...] = part

    @pl.when(kk > 0)
    def _acc():
        acc_ref[...] = acc_ref[...] + part

    @pl.when(kk == nk - 1)
    def _finish():
        t = acc_ref[...].T
        t = jnp.log(jnp.clip(t, EPS, None))
        for _ in range(SINKHORN_ITERS):
            t = t - _lse_tokens(t)
            t = t - _lse_experts(t)
        tt_ref[0] = t


def _scan_thresh(hist_v, histb_v, target, iota, start):
    def cond(s):
        c, total, _ = s
        return (total < target) & (c < NBCH)

    def body(s):
        c, total, bb = s
        base = (NBCH - 1 - c) * LANES
        h = hist_v[pl.ds(base, LANES)] + histb_v[pl.ds(base, LANES)]
        rh = jnp.flip(h)
        cs = jnp.cumsum(rh)
        idxvec = base + 15 - iota
        crossed = (total < target) & ((total + cs) >= target)
        combo = idxvec * 512 + (total + cs - rh)
        bc = jnp.max(jnp.where(crossed, combo, -1))
        return c + 1, total + jnp.sum(h), jnp.maximum(bb, bc)

    _, _, bb = jax.lax.while_loop(
        cond, body, (jnp.int32(start), jnp.int32(0), jnp.int32(-1)))
    return bb >> 9, bb & 511


def _topk_sc_kernel(tt_hbm, idx_hbm, gate_hbm,
                    vals_v, keys_v, hist_v, histb_v, ck_v, cv_v, gate_v,
                    *, ncol):
    cid = jax.lax.axis_index("c")
    sid = jax.lax.axis_index("s")
    wid = sid * 2 + cid
    iota = jax.lax.iota(jnp.int32, LANES)
    ones = jnp.ones((LANES,), jnp.int32)
    zeros_i = jnp.zeros((LANES,), jnp.int32)
    pad_k = jnp.full((LANES,), KEY_PAD, jnp.int32)
    cpw = ncol // 32

    def zero_hist(c, _):
        for u in range(4):
            hist_v[pl.ds((c * 4 + u) * LANES, LANES)] = zeros_i
            histb_v[pl.ds((c * 4 + u) * LANES, LANES)] = zeros_i
        return 0

    def key_at(c):
        v = vals_v[pl.ds(c * LANES, LANES)]
        bi = jax.lax.bitcast_convert_type(v, jnp.int32)
        return bi ^ ((bi >> 31) & jnp.int32(0x7FFFFFFF))

    for cc in range(cpw):
        col = wid * cpw + cc
        pltpu.sync_copy(tt_hbm.at[col], vals_v)

        jax.lax.fori_loop(0, NBCH // 4, zero_hist, 0)

        def p1(cu, _):
            c = cu * 2
            ku = key_at(c)
            keys_v[pl.ds(c * LANES, LANES)] = ku
            b = jax.lax.shift_right_logical(ku ^ jnp.int32(-(2 ** 31)), 21)
            plsc.addupdate_scatter(hist_v, [b], ones)
            ku2 = key_at(c + 1)
            keys_v[pl.ds((c + 1) * LANES, LANES)] = ku2
            b2_ = jax.lax.shift_right_logical(ku2 ^ jnp.int32(-(2 ** 31)), 21)
            plsc.addupdate_scatter(histb_v, [b2_], ones)
            return 0
        jax.lax.fori_loop(0, NCH // 2, p1, 0)

        b1, c_hi = _scan_thresh(hist_v, histb_v, jnp.int32(K), iota, 62)

        jax.lax.fori_loop(0, NBCH // 4, zero_hist, 0)

        def p2(cu, _):
            c = cu * 2
            ku = keys_v[pl.ds(c * LANES, LANES)]
            msk = jax.lax.shift_right_logical(
                ku ^ jnp.int32(-(2 ** 31)), 21) == b1
            sub = jax.lax.shift_right_logical(ku, 10) & jnp.int32(0x7FF)
            plsc.addupdate_scatter(hist_v, [sub], ones, mask=msk)
            ku2 = keys_v[pl.ds((c + 1) * LANES, LANES)]
            msk2 = jax.lax.shift_right_logical(
                ku2 ^ jnp.int32(-(2 ** 31)), 21) == b1
            sub2 = jax.lax.shift_right_logical(ku2, 10) & jnp.int32(0x7FF)
            plsc.addupdate_scatter(histb_v, [sub2], ones, mask=msk2)
            return 0
        jax.lax.fori_loop(0, NCH // 2, p2, 0)

        b2, _ = _scan_thresh(hist_v, histb_v, jnp.int32(K) - c_hi, iota, 0)
        thresh = ((b1 << 21) | (b2 << 10)) ^ jnp.int32(-(2 ** 31))

        def zc(c, _):
            for u in range(4):
                ck_v[pl.ds((c * 4 + u) * LANES, LANES)] = pad_k
                cv_v[pl.ds((c * 4 + u) * LANES, LANES)] = zeros_i
            return 0
        jax.lax.fori_loop(0, CAPB // 4, zc, 0)

        def pc(cu, off):
            for u in range(2):
                c = cu * 2 + u
                ku = keys_v[pl.ds(c * LANES, LANES)]
                msk = ku >= thresh
                iv = c * LANES + iota
                plsc.store_compressed(ck_v.at[pl.ds(off, LANES)], ku, mask=msk)
                plsc.store_compressed(cv_v.at[pl.ds(off, LANES)], iv, mask=msk)
                cnt = jnp.max(plsc.all_reduce_population_count(msk))
                off = jnp.minimum(off + cnt, CAP - LANES)
            return off
        off = jax.lax.fori_loop(0, NCH // 2, pc, jnp.int32(0))
        nb = jnp.minimum((off + LANES - 1) // LANES, CAPB)

        def sinit(i, _):
            kk = ck_v[pl.ds(i * LANES, LANES)]
            vv = cv_v[pl.ds(i * LANES, LANES)]
            kk, vv = plsc.sort_key_val(kk, vv, descending=True)
            ck_v[pl.ds(i * LANES, LANES)] = kk
            cv_v[pl.ds(i * LANES, LANES)] = vv
            return 0
        jax.lax.fori_loop(0, nb, sinit, 0)

        def rnd(r, _):
            par = r % 2

            def pair(p, _):
                i = par + 2 * p
                lo = i * LANES
                hi = lo + LANES
                ak = ck_v[pl.ds(lo, LANES)]
                av = cv_v[pl.ds(lo, LANES)]
                bk = jnp.flip(ck_v[pl.ds(hi, LANES)])
                bv = jnp.flip(cv_v[pl.ds(hi, LANES)])
                m = ak >= bk
                hk = jnp.where(m, ak, bk)
                hv = jnp.where(m, av, bv)
                lk = jnp.where(m, bk, ak)
                lv = jnp.where(m, bv, av)
                hk, hv = plsc.sort_key_val(hk, hv, descending=True)
                lk, lv = plsc.sort_key_val(lk, lv, descending=True)
                ck_v[pl.ds(lo, LANES)] = hk
                cv_v[pl.ds(lo, LANES)] = hv
                ck_v[pl.ds(hi, LANES)] = lk
                cv_v[pl.ds(hi, LANES)] = lv
                return 0
            jax.lax.fori_loop(0, (nb - par) // 2, pair, 0)
            return 0
        jax.lax.fori_loop(0, nb, rnd, 0)

        evens = iota * 2

        def repair(r, _):
            start = r % 2

            def rp(cu, _):
                for u in range(2):
                    i0 = start + (cu * 2 + u) * 32 + evens
                    i1 = i0 + 1
                    mvalid = i1 < CAP
                    k0 = plsc.load_gather(ck_v, [i0], mask=mvalid)
                    k1 = plsc.load_gather(ck_v, [i1], mask=mvalid)
                    v0 = plsc.load_gather(cv_v, [i0], mask=mvalid)
                    v1 = plsc.load_gather(cv_v, [i1], mask=mvalid)
                    cond = mvalid & (k0 == k1) & (v0 > v1)
                    plsc.store_scatter(cv_v, [i0], v1, mask=cond)
                    plsc.store_scatter(cv_v, [i1], v0, mask=cond)
                return 0
            jax.lax.fori_loop(0, CAP // 64, rp, 0)
            return 0
        jax.lax.fori_loop(0, 6, repair, 0)

        def pe(c, _):
            for u in range(4):
                kk = ck_v[pl.ds((c * 4 + u) * LANES, LANES)]
                g = jnp.where(kk > jnp.int32(KEY_HALF),
                              jnp.float32(1.0), jnp.float32(0.0))
                gate_v[pl.ds((c * 4 + u) * LANES, LANES)] = g
            return 0
        jax.lax.fori_loop(0, K // LANES // 4, pe, 0)

        pltpu.sync_copy(cv_v.at[pl.ds(0, K)], idx_hbm.at[col])
        pltpu.sync_copy(gate_v, gate_hbm.at[col])


@jax.jit
def kernel(x, to_gate_weight):
    b, n, d = x.shape
    e = to_gate_weight.shape[-1]
    k = n // e
    nk = 8
    dk = d // nk
    w = to_gate_weight[0]

    tt = pl.pallas_call(
        functools.partial(_gates_kernel, nk=nk),
        grid=(b, nk),
        in_specs=[
            pl.BlockSpec((1, n, dk), lambda bb, kk: (bb, 0, kk)),
            pl.BlockSpec((1, dk, e), lambda bb, kk: (0, kk, 0)),
        ],
        out_specs=pl.BlockSpec((1, e, n), lambda bb, kk: (bb, 0, 0)),
        out_shape=jax.ShapeDtypeStruct((b, e, n), jnp.float32),
        scratch_shapes=[pltpu.VMEM((n, e), jnp.float32)],
        compiler_params=pltpu.CompilerParams(
            dimension_semantics=("arbitrary", "arbitrary"),
        ),
    )(x, w[None])

    mesh = plsc.VectorSubcoreMesh(core_axis_name="c", subcore_axis_name="s")
    ncol = b * e
    topk = pl.kernel(
        functools.partial(_topk_sc_kernel, ncol=ncol),
        mesh=mesh,
        out_type=[
            jax.ShapeDtypeStruct((ncol, k), jnp.int32),
            jax.ShapeDtypeStruct((ncol, k), jnp.float32),
        ],
        scratch_types=[
            pltpu.VMEM((n,), jnp.float32),
            pltpu.VMEM((n,), jnp.int32),
            pltpu.VMEM((NBUCK,), jnp.int32),
            pltpu.VMEM((NBUCK,), jnp.int32),
            pltpu.VMEM((CAP,), jnp.int32),
            pltpu.VMEM((CAP,), jnp.int32),
            pltpu.VMEM((k,), jnp.float32),
        ],
        compiler_params=pltpu.CompilerParams(needs_layout_passes=False),
    )
    idx_f, gate_f = topk(tt.reshape(ncol, n))
    idx_out = jnp.swapaxes(idx_f.reshape(b, e, k), 1, 2)
    gate_out = jnp.swapaxes(gate_f.reshape(b, e, k), 1, 2)
    return idx_out, gate_out


# speedup vs baseline: 1.0028x; 1.0028x over previous
import functools
import math

import numpy as np
import jax
import jax.numpy as jnp
from jax.experimental import pallas as pl
from jax.experimental.pallas import tpu as pltpu
from jax.experimental.pallas import tpu_sc as plsc

SINKHORN_ITERS = 8
EPS = 1e-6
N = 4096
E = 16
K = 256
NCOL = 64
LANES = 16
NCH = N // LANES
NBUCK = 2048
NBCH = NBUCK // LANES
CAP = 320
CAPB = CAP // LANES

_HALF_BITS = int(np.float32(math.log(0.5)).view(np.uint32))
_KEY_HALF_U = (~_HALF_BITS) & 0xFFFFFFFF
KEY_HALF = (_KEY_HALF_U ^ 0x80000000) - (1 << 32) if (_KEY_HALF_U ^ 0x80000000) >= (1 << 31) else (_KEY_HALF_U ^ 0x80000000)
KEY_PAD = -(2 ** 31)


def _lse_tokens(t):
    m = jnp.max(t, axis=1, keepdims=True)
    return m + jnp.log(jnp.sum(jnp.exp(t - m), axis=1, keepdims=True))


def _lse_experts(t):
    m = jnp.max(t, axis=0, keepdims=True)
    return m + jnp.log(jnp.sum(jnp.exp(t - m), axis=0, keepdims=True))


def _gates_kernel(x_ref, w_ref, tt_ref, acc_ref, *, nk):
    kk = pl.program_id(1)

    part = jax.lax.dot_general(
        x_ref[0], w_ref[0],
        dimension_numbers=(((1,), (0,)), ((), ())),
        preferred_element_type=jnp.float32,
    )

    @pl.when(kk == 0)
    def _init():
        acc_ref[...] = part

    @pl.when(kk > 0)
    def _acc():
        acc_ref[...] = acc_ref[...] + part

    @pl.when(kk == nk - 1)
    def _finish():
        t = acc_ref[...].T
        t = jnp.log(jnp.clip(t, EPS, None))
        for _ in range(SINKHORN_ITERS):
            t = t - _lse_tokens(t)
            t = t - _lse_experts(t)
        tt_ref[0] = t


def _scan_thresh(hist_v, target, iota, start):
    def cond(s):
        c, total, _ = s
        return (total < target) & (c < NBCH)

    def body(s):
        c, total, bb = s
        base = (NBCH - 1 - c) * LANES
        h = hist_v[pl.ds(base, LANES)]
        rh = jnp.flip(h)
        cs = jnp.cumsum(rh)
        idxvec = base + 15 - iota
        crossed = (total < target) & ((total + cs) >= target)
        combo = idxvec * 512 + (total + cs - rh)
        bc = jnp.max(jnp.where(crossed, combo, -1))
        return c + 1, total + jnp.sum(h), jnp.maximum(bb, bc)

    _, _, bb = jax.lax.while_loop(
        cond, body, (jnp.int32(start), jnp.int32(0), jnp.int32(-1)))
    return bb >> 9, bb & 511


def _topk_sc_kernel(tt_hbm, idx_hbm, gate_hbm,
                    vals_v, keys_v, hist_v, ck_v, cv_v, gate_v,
                    *, ncol):
    cid = jax.lax.axis_index("c")
    sid = jax.lax.axis_index("s")
    wid = sid * 2 + cid
    iota = jax.lax.iota(jnp.int32, LANES)
    ones = jnp.ones((LANES,), jnp.int32)
    zeros_i = jnp.zeros((LANES,), jnp.int32)
    pad_k = jnp.full((LANES,), KEY_PAD, jnp.int32)
    cpw = ncol // 32

    def zero_hist(c, _):
        for u in range(4):
            hist_v[pl.ds((c * 4 + u) * LANES, LANES)] = zeros_i
        return 0

    def key_at(c):
        v = vals_v[pl.ds(c * LANES, LANES)]
        bi = jax.lax.bitcast_convert_type(v, jnp.int32)
        return bi ^ ((bi >> 31) & jnp.int32(0x7FFFFFFF))

    for cc in range(cpw):
        col = wid * cpw + cc
        pltpu.sync_copy(tt_hbm.at[col], vals_v)

        jax.lax.fori_loop(0, NBCH // 4, zero_hist, 0)

        def p1(c, _):
            ku = key_at(c)
            keys_v[pl.ds(c * LANES, LANES)] = ku
            b = jax.lax.shift_right_logical(ku ^ jnp.int32(-(2 ** 31)), 21)
            plsc.addupdate_scatter(hist_v, [b], ones)
            return 0
        jax.lax.fori_loop(0, NCH, p1, 0)

        b1, c_hi = _scan_thresh(hist_v, jnp.int32(K), iota, 62)

        jax.lax.fori_loop(0, NBCH // 4, zero_hist, 0)

        def p2(c, _):
            ku = keys_v[pl.ds(c * LANES, LANES)]
            msk = jax.lax.shift_right_logical(
                ku ^ jnp.int32(-(2 ** 31)), 21) == b1
            sub = jax.lax.shift_right_logical(ku, 10) & jnp.int32(0x7FF)
            plsc.addupdate_scatter(hist_v, [sub], ones, mask=msk)
            return 0
        jax.lax.fori_loop(0, NCH, p2, 0)

        b2, _ = _scan_thresh(hist_v, jnp.int32(K) - c_hi, iota, 0)
        thresh = ((b1 << 21) | (b2 << 10)) ^ jnp.int32(-(2 ** 31))

        def zc(c, _):
            for u in range(4):
                ck_v[pl.ds((c * 4 + u) * LANES, LANES)] = pad_k
                cv_v[pl.ds((c * 4 + u) * LANES, LANES)] = zeros_i
            return 0
        jax.lax.fori_loop(0, CAPB // 4, zc, 0)

        def pc(cu, off):
            for u in range(2):
                c = cu * 2 + u
                ku = keys_v[pl.ds(c * LANES, LANES)]
                msk = ku >= thresh
                iv = c * LANES + iota
                plsc.store_compressed(ck_v.at[pl.ds(off, LANES)], ku, mask=msk)
                plsc.store_compressed(cv_v.at[pl.ds(off, LANES)], iv, mask=msk)
                cnt = jnp.max(plsc.all_reduce_population_count(msk))
                off = jnp.minimum(off + cnt, CAP - LANES)
            return off
        off = jax.lax.fori_loop(0, NCH // 2, pc, jnp.int32(0))
        nb = jnp.minimum((off + LANES - 1) // LANES, CAPB)

        def sinit(i, _):
            kk = ck_v[pl.ds(i * LANES, LANES)]
            vv = cv_v[pl.ds(i * LANES, LANES)]
            kk, vv = plsc.sort_key_val(kk, vv, descending=True)
            ck_v[pl.ds(i * LANES, LANES)] = kk
            cv_v[pl.ds(i * LANES, LANES)] = vv
            return 0
        jax.lax.fori_loop(0, nb, sinit, 0)

        def rnd(r, _):
            par = r % 2

            def pair(p, _):
                i = par + 2 * p
                lo = i * LANES
                hi = lo + LANES
                ak = ck_v[pl.ds(lo, LANES)]
                av = cv_v[pl.ds(lo, LANES)]
                bk = jnp.flip(ck_v[pl.ds(hi, LANES)])
                bv = jnp.flip(cv_v[pl.ds(hi, LANES)])
                m = ak >= bk
                hk = jnp.where(m, ak, bk)
                hv = jnp.where(m, av, bv)
                lk = jnp.where(m, bk, ak)
                lv = jnp.where(m, bv, av)
                hk, hv = plsc.sort_key_val(hk, hv, descending=True)
                lk, lv = plsc.sort_key_val(lk, lv, descending=True)
                ck_v[pl.ds(lo, LANES)] = hk
                cv_v[pl.ds(lo, LANES)] = hv
                ck_v[pl.ds(hi, LANES)] = lk
                cv_v[pl.ds(hi, LANES)] = lv
                return 0
            jax.lax.fori_loop(0, (nb - par) // 2, pair, 0)
            return 0
        jax.lax.fori_loop(0, nb, rnd, 0)

        evens = iota * 2

        def repair(r, _):
            start = r % 2

            def rp(cu, _):
                for u in range(2):
                    i0 = start + (cu * 2 + u) * 32 + evens
                    i1 = i0 + 1
                    mvalid = i1 < CAP
                    k0 = plsc.load_gather(ck_v, [i0], mask=mvalid)
                    k1 = plsc.load_gather(ck_v, [i1], mask=mvalid)
                    v0 = plsc.load_gather(cv_v, [i0], mask=mvalid)
                    v1 = plsc.load_gather(cv_v, [i1], mask=mvalid)
                    cond = mvalid & (k0 == k1) & (v0 > v1)
                    plsc.store_scatter(cv_v, [i0], v1, mask=cond)
                    plsc.store_scatter(cv_v, [i1], v0, mask=cond)
                return 0
            jax.lax.fori_loop(0, CAP // 64, rp, 0)
            return 0
        jax.lax.fori_loop(0, 6, repair, 0)

        def pe(c, _):
            for u in range(4):
                kk = ck_v[pl.ds((c * 4 + u) * LANES, LANES)]
                g = jnp.where(kk > jnp.int32(KEY_HALF),
                              jnp.float32(1.0), jnp.float32(0.0))
                gate_v[pl.ds((c * 4 + u) * LANES, LANES)] = g
            return 0
        jax.lax.fori_loop(0, K // LANES // 4, pe, 0)

        pltpu.sync_copy(cv_v.at[pl.ds(0, K)], idx_hbm.at[col])
        pltpu.sync_copy(gate_v, gate_hbm.at[col])


@jax.jit
def kernel(x, to_gate_weight):
    b, n, d = x.shape
    e = to_gate_weight.shape[-1]
    k = n // e
    nk = 8
    dk = d // nk
    w = to_gate_weight[0]

    tt = pl.pallas_call(
        functools.partial(_gates_kernel, nk=nk),
        grid=(b, nk),
        in_specs=[
            pl.BlockSpec((1, n, dk), lambda bb, kk: (bb, 0, kk)),
            pl.BlockSpec((1, dk, e), lambda bb, kk: (0, kk, 0)),
        ],
        out_specs=pl.BlockSpec((1, e, n), lambda bb, kk: (bb, 0, 0)),
        out_shape=jax.ShapeDtypeStruct((b, e, n), jnp.float32),
        scratch_shapes=[pltpu.VMEM((n, e), jnp.float32)],
        compiler_params=pltpu.CompilerParams(
            dimension_semantics=("arbitrary", "arbitrary"),
        ),
    )(x, w[None])

    mesh = plsc.VectorSubcoreMesh(core_axis_name="c", subcore_axis_name="s")
    ncol = b * e
    topk = pl.kernel(
        functools.partial(_topk_sc_kernel, ncol=ncol),
        mesh=mesh,
        out_type=[
            jax.ShapeDtypeStruct((ncol, k), jnp.int32),
            jax.ShapeDtypeStruct((ncol, k), jnp.float32),
        ],
        scratch_types=[
            pltpu.VMEM((n,), jnp.float32),
            pltpu.VMEM((n,), jnp.int32),
            pltpu.VMEM((NBUCK,), jnp.int32),
            pltpu.VMEM((CAP,), jnp.int32),
            pltpu.VMEM((CAP,), jnp.int32),
            pltpu.VMEM((k,), jnp.float32),
        ],
        compiler_params=pltpu.CompilerParams(needs_layout_passes=False),
    )
    idx_f, gate_f = topk(tt.reshape(ncol, n))
    idx_out = jnp.swapaxes(idx_f.reshape(b, e, k), 1, 2)
    gate_out = jnp.swapaxes(gate_f.reshape(b, e, k), 1, 2)
    return idx_out, gate_out


# speedup vs baseline: 1.0521x; 1.0492x over previous
import functools
import math

import numpy as np
import jax
import jax.numpy as jnp
from jax.experimental import pallas as pl
from jax.experimental.pallas import tpu as pltpu
from jax.experimental.pallas import tpu_sc as plsc

SINKHORN_ITERS = 8
EPS = 1e-6
N = 4096
E = 16
K = 256
NCOL = 64
LANES = 16
NCH = N // LANES
NBUCK = 2048
NBCH = NBUCK // LANES
CAP = 320
CAPB = CAP // LANES

_HALF_BITS = int(np.float32(math.log(0.5)).view(np.uint32))
_KEY_HALF_U = (~_HALF_BITS) & 0xFFFFFFFF
KEY_HALF = (_KEY_HALF_U ^ 0x80000000) - (1 << 32) if (_KEY_HALF_U ^ 0x80000000) >= (1 << 31) else (_KEY_HALF_U ^ 0x80000000)
KEY_PAD = -(2 ** 31)


def _lse_tokens(t):
    m = jnp.max(t, axis=1, keepdims=True)
    return m + jnp.log(jnp.sum(jnp.exp(t - m), axis=1, keepdims=True))


def _lse_experts(t):
    m = jnp.max(t, axis=0, keepdims=True)
    return m + jnp.log(jnp.sum(jnp.exp(t - m), axis=0, keepdims=True))


def _gates_kernel(x_ref, w_ref, tt_ref, acc_ref, *, nk):
    kk = pl.program_id(1)

    part = jax.lax.dot_general(
        x_ref[0], w_ref[0],
        dimension_numbers=(((1,), (0,)), ((), ())),
        preferred_element_type=jnp.float32,
    )

    @pl.when(kk == 0)
    def _init():
        acc_ref[...] = part

    @pl.when(kk > 0)
    def _acc():
        acc_ref[...] = acc_ref[...] + part

    @pl.when(kk == nk - 1)
    def _finish():
        t = acc_ref[...].T
        t = jnp.log(jnp.clip(t, EPS, None))
        for _ in range(SINKHORN_ITERS):
            t = t - _lse_tokens(t)
            t = t - _lse_experts(t)
        tt_ref[0] = t


def _scan_thresh(hist_v, target, iota):
    def body(cu, carry):
        total, bb = carry
        for u in range(2):
            c = cu * 2 + u
            base = (NBCH - 1 - c) * LANES
            h = hist_v[pl.ds(base, LANES)]
            rh = jnp.flip(h)
            cs = jnp.cumsum(rh)
            idxvec = base + 15 - iota
            crossed = (total < target) & ((total + cs) >= target)
            combo = idxvec * 512 + (total + cs - rh)
            bc = jnp.max(jnp.where(crossed, combo, -1))
            total, bb = total + jnp.sum(h), jnp.maximum(bb, bc)
        return total, bb
    _, bb = jax.lax.fori_loop(
        0, NBCH // 2, body, (jnp.int32(0), jnp.int32(-1)))
    return bb >> 9, bb & 511


def _topk_sc_kernel(tt_hbm, idx_hbm, gate_hbm,
                    vals_v, keys_v, hist_v, ck_v, cv_v, gate_v,
                    *, ncol):
    cid = jax.lax.axis_index("c")
    sid = jax.lax.axis_index("s")
    wid = sid * 2 + cid
    iota = jax.lax.iota(jnp.int32, LANES)
    ones = jnp.ones((LANES,), jnp.int32)
    zeros_i = jnp.zeros((LANES,), jnp.int32)
    pad_k = jnp.full((LANES,), KEY_PAD, jnp.int32)
    cpw = ncol // 32

    def zero_hist(c, _):
        for u in range(4):
            hist_v[pl.ds((c * 4 + u) * LANES, LANES)] = zeros_i
        return 0

    def key_at(c):
        v = vals_v[pl.ds(c * LANES, LANES)]
        bi = jax.lax.bitcast_convert_type(v, jnp.int32)
        return bi ^ ((bi >> 31) & jnp.int32(0x7FFFFFFF))

    for cc in range(cpw):
        col = wid * cpw + cc
        pltpu.sync_copy(tt_hbm.at[col], vals_v)

        jax.lax.fori_loop(0, NBCH // 4, zero_hist, 0)

        def p1(c, _):
            ku = key_at(c)
            keys_v[pl.ds(c * LANES, LANES)] = ku
            b = jax.lax.shift_right_logical(ku ^ jnp.int32(-(2 ** 31)), 21)
            plsc.addupdate_scatter(hist_v, [b], ones)
            return 0
        jax.lax.fori_loop(0, NCH, p1, 0)

        b1, c_hi = _scan_thresh(hist_v, jnp.int32(K), iota)

        jax.lax.fori_loop(0, NBCH // 4, zero_hist, 0)

        def p2(c, _):
            ku = keys_v[pl.ds(c * LANES, LANES)]
            msk = jax.lax.shift_right_logical(
                ku ^ jnp.int32(-(2 ** 31)), 21) == b1
            sub = jax.lax.shift_right_logical(ku, 10) & jnp.int32(0x7FF)
            plsc.addupdate_scatter(hist_v, [sub], ones, mask=msk)
            return 0
        jax.lax.fori_loop(0, NCH, p2, 0)

        b2, _ = _scan_thresh(hist_v, jnp.int32(K) - c_hi, iota)
        thresh = ((b1 << 21) | (b2 << 10)) ^ jnp.int32(-(2 ** 31))

        def zc(c, _):
            for u in range(4):
                ck_v[pl.ds((c * 4 + u) * LANES, LANES)] = pad_k
                cv_v[pl.ds((c * 4 + u) * LANES, LANES)] = zeros_i
            return 0
        jax.lax.fori_loop(0, CAPB // 4, zc, 0)

        def pc(cu, off):
            for u in range(2):
                c = cu * 2 + u
                ku = keys_v[pl.ds(c * LANES, LANES)]
                msk = ku >= thresh
                iv = c * LANES + iota
                plsc.store_compressed(ck_v.at[pl.ds(off, LANES)], ku, mask=msk)
                plsc.store_compressed(cv_v.at[pl.ds(off, LANES)], iv, mask=msk)
                cnt = jnp.max(plsc.all_reduce_population_count(msk))
                off = jnp.minimum(off + cnt, CAP - LANES)
            return off
        off = jax.lax.fori_loop(0, NCH // 2, pc, jnp.int32(0))
        nb = jnp.minimum((off + LANES - 1) // LANES, CAPB)

        def sinit(i, _):
            kk = ck_v[pl.ds(i * LANES, LANES)]
            vv = cv_v[pl.ds(i * LANES, LANES)]
            kk, vv = plsc.sort_key_val(kk, vv, descending=True)
            ck_v[pl.ds(i * LANES, LANES)] = kk
            cv_v[pl.ds(i * LANES, LANES)] = vv
            return 0
        jax.lax.fori_loop(0, nb, sinit, 0)

        def rnd(r, _):
            par = r % 2

            def pair(p, _):
                i = par + 2 * p
                lo = i * LANES
                hi = lo + LANES
                ak = ck_v[pl.ds(lo, LANES)]
                av = cv_v[pl.ds(lo, LANES)]
                bk = jnp.flip(ck_v[pl.ds(hi, LANES)])
                bv = jnp.flip(cv_v[pl.ds(hi, LANES)])
                m = ak >= bk
                hk = jnp.where(m, ak, bk)
                hv = jnp.where(m, av, bv)
                lk = jnp.where(m, bk, ak)
                lv = jnp.where(m, bv, av)
                hk, hv = plsc.sort_key_val(hk, hv, descending=True)
                lk, lv = plsc.sort_key_val(lk, lv, descending=True)
                ck_v[pl.ds(lo, LANES)] = hk
                cv_v[pl.ds(lo, LANES)] = hv
                ck_v[pl.ds(hi, LANES)] = lk
                cv_v[pl.ds(hi, LANES)] = lv
                return 0
            jax.lax.fori_loop(0, (nb - par) // 2, pair, 0)
            return 0
        jax.lax.fori_loop(0, nb, rnd, 0)

        evens = iota * 2

        def repair(r, _):
            start = r % 2

            def rp(cu, _):
                for u in range(2):
                    i0 = start + (cu * 2 + u) * 32 + evens
                    i1 = i0 + 1
                    mvalid = i1 < CAP
                    k0 = plsc.load_gather(ck_v, [i0], mask=mvalid)
                    k1 = plsc.load_gather(ck_v, [i1], mask=mvalid)
                    v0 = plsc.load_gather(cv_v, [i0], mask=mvalid)
                    v1 = plsc.load_gather(cv_v, [i1], mask=mvalid)
                    cond = mvalid & (k0 == k1) & (v0 > v1)
                    plsc.store_scatter(cv_v, [i0], v1, mask=cond)
                    plsc.store_scatter(cv_v, [i1], v0, mask=cond)
                return 0
            jax.lax.fori_loop(0, CAP // 64, rp, 0)
            return 0
        jax.lax.fori_loop(0, 6, repair, 0)

        def pe(c, _):
            for u in range(4):
                kk = ck_v[pl.ds((c * 4 + u) * LANES, LANES)]
                g = jnp.where(kk > jnp.int32(KEY_HALF),
                              jnp.float32(1.0), jnp.float32(0.0))
                gate_v[pl.ds((c * 4 + u) * LANES, LANES)] = g
            return 0
        jax.lax.fori_loop(0, K // LANES // 4, pe, 0)

        pltpu.sync_copy(cv_v.at[pl.ds(0, K)], idx_hbm.at[col])
        pltpu.sync_copy(gate_v, gate_hbm.at[col])


@jax.jit
def kernel(x, to_gate_weight):
    b, n, d = x.shape
    e = to_gate_weight.shape[-1]
    k = n // e
    nk = 8
    dk = d // nk
    w = to_gate_weight[0]

    tt = pl.pallas_call(
        functools.partial(_gates_kernel, nk=nk),
        grid=(b, nk),
        in_specs=[
            pl.BlockSpec((1, n, dk), lambda bb, kk: (bb, 0, kk)),
            pl.BlockSpec((1, dk, e), lambda bb, kk: (0, kk, 0)),
        ],
        out_specs=pl.BlockSpec((1, e, n), lambda bb, kk: (bb, 0, 0)),
        out_shape=jax.ShapeDtypeStruct((b, e, n), jnp.float32),
        scratch_shapes=[pltpu.VMEM((n, e), jnp.float32)],
        compiler_params=pltpu.CompilerParams(
            dimension_semantics=("arbitrary", "arbitrary"),
        ),
    )(x, w[None])

    mesh = plsc.VectorSubcoreMesh(core_axis_name="c", subcore_axis_name="s")
    ncol = b * e
    topk = pl.kernel(
        functools.partial(_topk_sc_kernel, ncol=ncol),
        mesh=mesh,
        out_type=[
            jax.ShapeDtypeStruct((ncol, k), jnp.int32),
            jax.ShapeDtypeStruct((ncol, k), jnp.float32),
        ],
        scratch_types=[
            pltpu.VMEM((n,), jnp.float32),
            pltpu.VMEM((n,), jnp.int32),
            pltpu.VMEM((NBUCK,), jnp.int32),
            pltpu.VMEM((CAP,), jnp.int32),
            pltpu.VMEM((CAP,), jnp.int32),
            pltpu.VMEM((k,), jnp.float32),
        ],
        compiler_params=pltpu.CompilerParams(needs_layout_passes=False),
    )
    idx_f, gate_f = topk(tt.reshape(ncol, n))
    idx_out = jnp.swapaxes(idx_f.reshape(b, e, k), 1, 2)
    gate_out = jnp.swapaxes(gate_f.reshape(b, e, k), 1, 2)
    return idx_out, gate_out


# speedup vs baseline: 1.0540x; 1.0018x over previous
import functools
import math

import numpy as np
import jax
import jax.numpy as jnp
from jax.experimental import pallas as pl
from jax.experimental.pallas import tpu as pltpu
from jax.experimental.pallas import tpu_sc as plsc

SINKHORN_ITERS = 8
EPS = 1e-6
N = 4096
K = 256
LANES = 16
NCH = N // LANES
NBUCK = 2048
NBCH = NBUCK // LANES
CAP = 320
CAPB = CAP // LANES

_HALF_BITS = int(np.float32(math.log(0.5)).view(np.uint32))
_KEY_HALF_U = (~_HALF_BITS) & 0xFFFFFFFF
KEY_HALF = (_KEY_HALF_U ^ 0x80000000) - (1 << 32) if (_KEY_HALF_U ^ 0x80000000) >= (1 << 31) else (_KEY_HALF_U ^ 0x80000000)
KEY_PAD = -(2 ** 31)


def _lse_tokens(t):
    m = jnp.max(t, axis=1, keepdims=True)
    return m + jnp.log(jnp.sum(jnp.exp(t - m), axis=1, keepdims=True))


def _lse_experts(t):
    m = jnp.max(t, axis=0, keepdims=True)
    return m + jnp.log(jnp.sum(jnp.exp(t - m), axis=0, keepdims=True))


def _gates_kernel(x_ref, w_ref, tt_ref, acc_ref, *, nk):
    kk = pl.program_id(1)

    part = jax.lax.dot_general(
        x_ref[0], w_ref[0],
        dimension_numbers=(((1,), (0,)), ((), ())),
        preferred_element_type=jnp.float32,
    )

    @pl.when(kk == 0)
    def _init():
        acc_ref[...] = part

    @pl.when(kk > 0)
    def _acc():
        acc_ref[...] = acc_ref[...] + part

    @pl.when(kk == nk - 1)
    def _finish():
        t = acc_ref[...].T
        t = jnp.log(jnp.clip(t, EPS, None))
        for _ in range(SINKHORN_ITERS):
            t = t - _lse_tokens(t)
            t = t - _lse_experts(t)
        tt_ref[0] = t


def _scan_thresh(hist_v, target, iota):
    def body(cu, carry):
        total, bb = carry
        for u in range(2):
            c = cu * 2 + u
            base = (NBCH - 1 - c) * LANES
            h = hist_v[pl.ds(base, LANES)]
            rh = jnp.flip(h)
            cs = jnp.cumsum(rh)
            idxvec = base + 15 - iota
            crossed = (total < target) & ((total + cs) >= target)
            combo = idxvec * 512 + (total + cs - rh)
            bc = jnp.max(jnp.where(crossed, combo, -1))
            total, bb = total + jnp.sum(h), jnp.maximum(bb, bc)
        return total, bb
    _, bb = jax.lax.fori_loop(
        0, NBCH // 2, body, (jnp.int32(0), jnp.int32(-1)))
    return bb >> 9, bb & 511


def _topk_sc_kernel(tt_hbm, idx_hbm, gate_hbm,
                    vals_v, keys_v, hist_v, ck_v, cv_v, gate_v,
                    *, ncol):
    cid = jax.lax.axis_index("c")
    sid = jax.lax.axis_index("s")
    wid = sid * 2 + cid
    iota = jax.lax.iota(jnp.int32, LANES)
    ones = jnp.ones((LANES,), jnp.int32)
    zeros_i = jnp.zeros((LANES,), jnp.int32)
    pad_k = jnp.full((LANES,), KEY_PAD, jnp.int32)
    cpw = ncol // 32

    def zero_hist(c, _):
        for u in range(4):
            hist_v[pl.ds((c * 4 + u) * LANES, LANES)] = zeros_i
        return 0

    def key_at(c):
        v = vals_v[pl.ds(c * LANES, LANES)]
        bi = jax.lax.bitcast_convert_type(v, jnp.int32)
        return bi ^ ((bi >> 31) & jnp.int32(0x7FFFFFFF))

    for cc in range(cpw):
        col = wid * cpw + cc
        pltpu.sync_copy(tt_hbm.at[col], vals_v)

        jax.lax.fori_loop(0, NBCH // 4, zero_hist, 0)

        def p1(c, _):
            ku = key_at(c)
            keys_v[pl.ds(c * LANES, LANES)] = ku
            b = jax.lax.shift_right_logical(ku ^ jnp.int32(-(2 ** 31)), 21)
            plsc.addupdate_scatter(hist_v, [b], ones)
            return 0
        jax.lax.fori_loop(0, NCH, p1, 0)

        b1, c_hi = _scan_thresh(hist_v, jnp.int32(K), iota)

        jax.lax.fori_loop(0, NBCH // 4, zero_hist, 0)

        def p2(c, _):
            ku = keys_v[pl.ds(c * LANES, LANES)]
            msk = jax.lax.shift_right_logical(
                ku ^ jnp.int32(-(2 ** 31)), 21) == b1
            sub = jax.lax.shift_right_logical(ku, 10) & jnp.int32(0x7FF)
            plsc.addupdate_scatter(hist_v, [sub], ones, mask=msk)
            return 0
        jax.lax.fori_loop(0, NCH, p2, 0)

        b2, _ = _scan_thresh(hist_v, jnp.int32(K) - c_hi, iota)
        thresh = ((b1 << 21) | (b2 << 10)) ^ jnp.int32(-(2 ** 31))

        def zc(c, _):
            for u in range(4):
                ck_v[pl.ds((c * 4 + u) * LANES, LANES)] = pad_k
                cv_v[pl.ds((c * 4 + u) * LANES, LANES)] = zeros_i
            return 0
        jax.lax.fori_loop(0, CAPB // 4, zc, 0)

        def pc(cu, off):
            for u in range(2):
                c = cu * 2 + u
                ku = keys_v[pl.ds(c * LANES, LANES)]
                msk = ku >= thresh
                iv = c * LANES + iota
                plsc.store_compressed(ck_v.at[pl.ds(off, LANES)], ku, mask=msk)
                plsc.store_compressed(cv_v.at[pl.ds(off, LANES)], iv, mask=msk)
                cnt = jnp.max(plsc.all_reduce_population_count(msk))
                off = jnp.minimum(off + cnt, CAP - LANES)
            return off
        off = jax.lax.fori_loop(0, NCH // 2, pc, jnp.int32(0))
        nb = jnp.minimum((off + LANES - 1) // LANES, CAPB)

        def sinit(i, _):
            kk = ck_v[pl.ds(i * LANES, LANES)]
            vv = cv_v[pl.ds(i * LANES, LANES)]
            kk, vv = plsc.sort_key_val(kk, vv, descending=True)
            ck_v[pl.ds(i * LANES, LANES)] = kk
            cv_v[pl.ds(i * LANES, LANES)] = vv
            return 0
        jax.lax.fori_loop(0, nb, sinit, 0)

        def rnd(r, _):
            par = r % 2

            def pair(p, _):
                i = par + 2 * p
                lo = i * LANES
                hi = lo + LANES
                ak = ck_v[pl.ds(lo, LANES)]
                av = cv_v[pl.ds(lo, LANES)]
                bk = jnp.flip(ck_v[pl.ds(hi, LANES)])
                bv = jnp.flip(cv_v[pl.ds(hi, LANES)])
                m = ak >= bk
                hk = jnp.where(m, ak, bk)
                hv = jnp.where(m, av, bv)
                lk = jnp.where(m, bk, ak)
                lv = jnp.where(m, bv, av)
                hk, hv = plsc.sort_key_val(hk, hv, descending=True)
                lk, lv = plsc.sort_key_val(lk, lv, descending=True)
                ck_v[pl.ds(lo, LANES)] = hk
                cv_v[pl.ds(lo, LANES)] = hv
                ck_v[pl.ds(hi, LANES)] = lk
                cv_v[pl.ds(hi, LANES)] = lv
                return 0
            jax.lax.fori_loop(0, (nb - par) // 2, pair, 0)
            return 0
        jax.lax.fori_loop(0, nb, rnd, 0)

        evens = iota * 2

        def repair(r, _):
            start = r % 2

            def rp(cu, _):
                for u in range(2):
                    i0 = start + (cu * 2 + u) * 32 + evens
                    i1 = i0 + 1
                    mvalid = i1 < CAP
                    k0 = plsc.load_gather(ck_v, [i0], mask=mvalid)
                    k1 = plsc.load_gather(ck_v, [i1], mask=mvalid)
                    v0 = plsc.load_gather(cv_v, [i0], mask=mvalid)
                    v1 = plsc.load_gather(cv_v, [i1], mask=mvalid)
                    cond = mvalid & (k0 == k1) & (v0 > v1)
                    plsc.store_scatter(cv_v, [i0], v1, mask=cond)
                    plsc.store_scatter(cv_v, [i1], v0, mask=cond)
                return 0
            jax.lax.fori_loop(0, CAP // 64, rp, 0)
            return 0
        jax.lax.fori_loop(0, 6, repair, 0)

        def pe(c, _):
            for u in range(4):
                kk = ck_v[pl.ds((c * 4 + u) * LANES, LANES)]
                g = jnp.where(kk > jnp.int32(KEY_HALF),
                              jnp.float32(1.0), jnp.float32(0.0))
                gate_v[pl.ds((c * 4 + u) * LANES, LANES)] = g
            return 0
        jax.lax.fori_loop(0, K // LANES // 4, pe, 0)

        pltpu.sync_copy(cv_v.at[pl.ds(0, K)], idx_hbm.at[col])
        pltpu.sync_copy(gate_v, gate_hbm.at[col])


@jax.jit
def kernel(x, to_gate_weight):
    b, n, d = x.shape
    e = to_gate_weight.shape[-1]
    k = n // e
    nk = 8
    dk = d // nk
    w = to_gate_weight[0]

    tt = pl.pallas_call(
        functools.partial(_gates_kernel, nk=nk),
        grid=(b, nk),
        in_specs=[
            pl.BlockSpec((1, n, dk), lambda bb, kk: (bb, 0, kk)),
            pl.BlockSpec((1, dk, e), lambda bb, kk: (0, kk, 0)),
        ],
        out_specs=pl.BlockSpec((1, e, n), lambda bb, kk: (bb, 0, 0)),
        out_shape=jax.ShapeDtypeStruct((b, e, n), jnp.float32),
        scratch_shapes=[pltpu.VMEM((n, e), jnp.float32)],
        compiler_params=pltpu.CompilerParams(
            dimension_semantics=("arbitrary", "arbitrary"),
        ),
    )(x, w[None])

    mesh = plsc.VectorSubcoreMesh(core_axis_name="c", subcore_axis_name="s")
    ncol = b * e
    topk = pl.kernel(
        functools.partial(_topk_sc_kernel, ncol=ncol),
        mesh=mesh,
        out_type=[
            jax.ShapeDtypeStruct((ncol, k), jnp.int32),
            jax.ShapeDtypeStruct((ncol, k), jnp.float32),
        ],
        scratch_types=[
            pltpu.VMEM((n,), jnp.float32),
            pltpu.VMEM((n,), jnp.int32),
            pltpu.VMEM((NBUCK,), jnp.int32),
            pltpu.VMEM((CAP,), jnp.int32),
            pltpu.VMEM((CAP,), jnp.int32),
            pltpu.VMEM((k,), jnp.float32),
        ],
        compiler_params=pltpu.CompilerParams(needs_layout_passes=False),
    )
    idx_f, gate_f = topk(tt.reshape(ncol, n))
    idx_out = jnp.swapaxes(idx_f.reshape(b, e, k), 1, 2)
    gate_out = jnp.swapaxes(gate_f.reshape(b, e, k), 1, 2)
    return idx_out, gate_out
